# baseline (speedup 1.0000x reference)
import dataclasses
import functools

import jax
import jax.numpy as jnp
from jax import lax
from jax.experimental import pallas as pl
from jax.experimental.pallas import tpu as pltpu
from jax.experimental.pallas import tpu_sc as plsc

D = 128
NC = 2
NS = 16
NW = NC * NS
LANES = 16
CH = 128
NBUF = 2
NRCW = 4
BR = 1000


def _sc_compiler_params():
    cp = pltpu.CompilerParams()
    if "needs_layout_passes" in pltpu.CompilerParams.__dataclass_fields__:
        cp = dataclasses.replace(cp, needs_layout_passes=False)
    return cp


def _make_deg_kernel(n, nch):
    mesh = plsc.VectorSubcoreMesh(core_axis_name="c", subcore_axis_name="s")

    @functools.partial(
        pl.kernel, mesh=mesh,
        out_type=jax.ShapeDtypeStruct((NW, 1, n), jnp.float32),
        compiler_params=_sc_compiler_params(),
        scratch_types=[
            pltpu.VMEM((nch, CH), jnp.int32),
            pltpu.VMEM((nch, CH), jnp.float32),
            pltpu.VMEM((n,), jnp.float32),
        ],
    )
    def deg_kernel(col_hbm, ew_hbm, out_hbm, colv, ewv, acc):
        wid = lax.axis_index("s") * NC + lax.axis_index("c")
        pltpu.sync_copy(col_hbm.at[wid], colv)
        pltpu.sync_copy(ew_hbm.at[wid], ewv)
        zeros = jnp.zeros((LANES,), jnp.float32)

        @pl.loop(0, n // LANES)
        def _(i):
            acc[pl.ds(i * LANES, LANES)] = zeros

        @pl.loop(0, nch)
        def _(g):
            for k in range(CH // LANES):
                idx = colv[g, pl.ds(k * LANES, LANES)]
                vals = ewv[g, pl.ds(k * LANES, LANES)]
                plsc.addupdate_scatter(acc, [idx], vals)

        pltpu.sync_copy(acc, out_hbm.at[wid, 0])

    return deg_kernel


def _make_edge_kernel(n, nch_a, nch_b):
    rps = (n // (NS * 8)) * 8
    rem = n - NS * rps
    assert rem % 8 == 0 and rem <= CH
    assert min(nch_a, nch_b) >= 4
    full, tail = divmod(rps, CH)
    mesh = plsc.VectorSubcoreMesh(core_axis_name="c", subcore_axis_name="s")

    @functools.partial(
        pl.kernel, mesh=mesh,
        out_type=jax.ShapeDtypeStruct((NC, n, D), jnp.float32),
        compiler_params=_sc_compiler_params(),
        scratch_types=[
            pltpu.VMEM((NRCW * 3, CH), jnp.int32),
            pltpu.VMEM((NBUF, CH, D), jnp.float32),
            pltpu.VMEM_SHARED((n, D), jnp.float32),
            pltpu.SemaphoreType.DMA((NRCW + 2 * NBUF,)),
        ],
    )
    def edge_kernel(y_hbm, rcw_hbm, out_hbm, rcw, rows, acc, sem):
        rsem = sem.at[pl.ds(0, NRCW)]
        gsem = sem.at[pl.ds(NRCW, NBUF)]
        ssem = sem.at[pl.ds(NRCW + NBUF, NBUF)]
        cid = lax.axis_index("c")
        sid = lax.axis_index("s")
        mynch = jnp.where(cid == 0, nch_a, nch_b)
        cbase = jnp.where(cid == 0, sid * nch_a, NS * nch_a + sid * nch_b)

        def r_start(g):
            b = lax.rem(g, NRCW)
            pltpu.async_copy(rcw_hbm.at[cbase + g], rcw.at[pl.ds(b * 3, 3)],
                             rsem.at[b])

        def r_wait(g):
            b = lax.rem(g, NRCW)
            pltpu.make_async_copy(rcw_hbm.at[cbase + g],
                                  rcw.at[pl.ds(b * 3, 3)],
                                  rsem.at[b]).wait()

        def g_start(g):
            b = lax.rem(g, NBUF)
            b5 = lax.rem(g, NRCW)
            pltpu.async_copy(y_hbm.at[rcw.at[b5 * 3]], rows.at[b], gsem.at[b])

        def g_wait(g):
            b = lax.rem(g, NBUF)
            b5 = lax.rem(g, NRCW)
            pltpu.make_async_copy(y_hbm.at[rcw.at[b5 * 3]], rows.at[b],
                                  gsem.at[b]).wait()

        def s_start(g):
            b = lax.rem(g, NBUF)
            b5 = lax.rem(g, NRCW)
            pltpu.async_copy(rows.at[b], acc.at[rcw.at[b5 * 3 + 1]],
                             ssem.at[b], add=True)

        def s_wait(g):
            b = lax.rem(g, NBUF)
            b5 = lax.rem(g, NRCW)
            pltpu.make_async_copy(rows.at[b], acc.at[rcw.at[b5 * 3 + 1]],
                                  ssem.at[b]).wait()

        def scale(g):
            b = lax.rem(g, NBUF)
            ewrow = lax.rem(g, NRCW) * 3 + 2

            @pl.loop(0, CH, unroll=4)
            def _(i):
                ri = jnp.full((LANES,), ewrow, jnp.int32)
                ii = jnp.full((LANES,), i, jnp.int32)
                w = plsc.bitcast(plsc.load_gather(rcw, [ri, ii]),
                                 jnp.float32)
                for j in range(D // LANES):
                    sl = (b, i, pl.ds(j * LANES, LANES))
                    rows[sl] = rows[sl] * w

        zeros = jnp.zeros((LANES,), jnp.float32)

        @pl.loop(0, CH)
        def _(i):
            for j in range(D // LANES):
                rows[0, i, pl.ds(j * LANES, LANES)] = zeros

        base = sid * rps

        @pl.loop(0, full)
        def _(t):
            pltpu.sync_copy(rows.at[0], acc.at[pl.ds(base + t * CH, CH)])

        if tail:
            pltpu.sync_copy(rows.at[0, pl.ds(0, tail)],
                            acc.at[pl.ds(base + full * CH, tail)])
        if rem:
            @pl.when(sid == 0)
            def _():
                pltpu.sync_copy(rows.at[0, pl.ds(0, rem)],
                                acc.at[pl.ds(NS * rps, rem)])

        plsc.subcore_barrier()

        r_start(0)
        r_start(1)

        @pl.loop(0, max(nch_a, nch_b) + 1)
        def _(t0):
            @pl.when(t0 < mynch)
            def _():
                @pl.when(t0 >= NBUF)
                def _():
                    s_wait(t0 - NBUF)

                @pl.when(jnp.logical_and(t0 + 1 >= 2, t0 + 1 < mynch))
                def _():
                    r_start(t0 + 1)

                r_wait(t0)
                g_start(t0)

            @pl.when(jnp.logical_and(t0 >= 1, t0 <= mynch))
            def _():
                t = t0 - 1
                g_wait(t)
                scale(t)
                s_start(t)

        @pl.loop(0, NBUF)
        def _(k):
            s_wait(mynch - NBUF + k)

        plsc.subcore_barrier()

        pltpu.sync_copy(acc.at[pl.ds(base, rps)],
                        out_hbm.at[cid, pl.ds(base, rps)])
        if rem:
            @pl.when(sid == 0)
            def _():
                pltpu.sync_copy(acc.at[pl.ds(NS * rps, rem)],
                                out_hbm.at[cid, pl.ds(NS * rps, rem)])

    return edge_kernel


def _ln_gelu(t, g, b):
    mu = jnp.mean(t, axis=-1, keepdims=True)
    var = jnp.mean((t - mu) ** 2, axis=-1, keepdims=True)
    t = (t - mu) * lax.rsqrt(var + 1e-5) * g + b
    return 0.5 * t * (1.0 + lax.erf(t * 0.7071067811865476))


def _dis_body(degp_ref, dis_ref):
    deg = 1.0 + jnp.sum(degp_ref[...], axis=0, keepdims=True)
    dis_ref[...] = jnp.where(deg > 0, lax.rsqrt(jnp.maximum(deg, 1e-30)), 0.0)


def _k0_body(x_ref, lw_ref, lb_ref, g_ref, b_ref, w1_ref, dis_ref,
             h_ref, y_ref):
    h = jnp.dot(x_ref[...], lw_ref[...],
                preferred_element_type=jnp.float32) + lb_ref[...]
    h = _ln_gelu(h, g_ref[...], b_ref[...])
    h_ref[...] = h
    y_ref[...] = dis_ref[...] * jnp.dot(h, w1_ref[...],
                                        preferred_element_type=jnp.float32)


def _kmid_body(acc_ref, y_ref, hp_ref, dis_ref, cb_ref, g_ref, b_ref, wn_ref,
               h_ref, yo_ref):
    s = acc_ref[0] + acc_ref[1] + y_ref[...]
    conv = dis_ref[...] * s + cb_ref[...]
    h = _ln_gelu(conv, g_ref[...], b_ref[...]) + hp_ref[...]
    h_ref[...] = h
    yo_ref[...] = dis_ref[...] * jnp.dot(h, wn_ref[...],
                                         preferred_element_type=jnp.float32)


def _klast_body(acc_ref, y_ref, hp_ref, dis_ref, cb_ref, g_ref, b_ref, h_ref):
    s = acc_ref[0] + acc_ref[1] + y_ref[...]
    conv = dis_ref[...] * s + cb_ref[...]
    h_ref[...] = _ln_gelu(conv, g_ref[...], b_ref[...]) + hp_ref[...]


def _row_spec(n):
    return pl.BlockSpec((BR, D), lambda g: (g, 0))


_W_SPEC = pl.BlockSpec((D, D), lambda g: (0, 0))
_P_SPEC = pl.BlockSpec((1, D), lambda g: (0, 0))


def _tc_k0(n, x, lw, lb, g0, b0, w1, dis):
    sds = jax.ShapeDtypeStruct((n, D), jnp.float32)
    return pl.pallas_call(
        _k0_body,
        grid=(n // BR,),
        in_specs=[_row_spec(n), _W_SPEC, _P_SPEC, _P_SPEC, _P_SPEC, _W_SPEC,
                  pl.BlockSpec((BR, 1), lambda g: (g, 0))],
        out_specs=[_row_spec(n), _row_spec(n)],
        out_shape=[sds, sds],
    )(x, lw, lb, g0, b0, w1, dis)


def _tc_mid(n, acc, y, hp, dis, cb, g, b, wn):
    sds = jax.ShapeDtypeStruct((n, D), jnp.float32)
    return pl.pallas_call(
        _kmid_body,
        grid=(n // BR,),
        in_specs=[pl.BlockSpec((NC, BR, D), lambda g: (0, g, 0)),
                  _row_spec(n), _row_spec(n),
                  pl.BlockSpec((BR, 1), lambda g: (g, 0)),
                  _P_SPEC, _P_SPEC, _P_SPEC, _W_SPEC],
        out_specs=[_row_spec(n), _row_spec(n)],
        out_shape=[sds, sds],
    )(acc, y, hp, dis, cb, g, b, wn)


def _tc_last(n, acc, y, hp, dis, cb, g, b):
    sds = jax.ShapeDtypeStruct((n, D), jnp.float32)
    return pl.pallas_call(
        _klast_body,
        grid=(n // BR,),
        in_specs=[pl.BlockSpec((NC, BR, D), lambda g: (0, g, 0)),
                  _row_spec(n), _row_spec(n),
                  pl.BlockSpec((BR, 1), lambda g: (g, 0)),
                  _P_SPEC, _P_SPEC, _P_SPEC],
        out_specs=_row_spec(n),
        out_shape=sds,
    )(acc, y, hp, dis, cb, g, b)


def _tc_dis(n, degp):
    return pl.pallas_call(
        _dis_body,
        out_shape=jax.ShapeDtypeStruct((1, n), jnp.float32),
    )(degp)


def kernel(x, edge_index, edge_weight, params):
    n, d = x.shape
    e = edge_weight.shape[0]
    assert d == D and n % LANES == 0 and n % BR == 0

    row = edge_index[0].astype(jnp.int32)
    col = edge_index[1].astype(jnp.int32)
    ew = edge_weight.astype(jnp.float32)

    epw = -(-e // NW)
    epw = -(-epw // CH) * CH
    nchd = epw // CH
    padd = epw * NW - e
    colp = jnp.pad(col, (0, padd)).reshape(NW, nchd, CH)
    ewpd = jnp.pad(ew, (0, padd)).reshape(NW, nchd, CH)
    degp = _make_deg_kernel(n, nchd)(colp, ewpd).reshape(NW, n)
    dis = _tc_dis(n, degp).reshape(n, 1)

    tpp = -(-(-(-e // CH)) // NS)
    nch_a = max(NBUF + 2, round(tpp * 0.186))
    nch_b = tpp - nch_a
    tot = NS * tpp
    pad = tot * CH - e
    rowp = jnp.pad(row, (0, pad)).reshape(tot, 1, CH)
    colp2 = jnp.pad(col, (0, pad)).reshape(tot, 1, CH)
    ewp = jnp.pad(ew, (0, pad)).reshape(tot, 1, CH)
    rcw = jnp.concatenate(
        [rowp, colp2, lax.bitcast_convert_type(ewp, jnp.int32)], axis=1)

    p = params
    lb = p["lin_b"].reshape(1, D)
    ln_g = [p["ln_g"][i].reshape(1, D) for i in range(7)]
    ln_b = [p["ln_b"][i].reshape(1, D) for i in range(7)]
    cb = [p["conv_b"][i].reshape(1, D) for i in range(6)]
    cw = [p["conv_W"][i] for i in range(6)]

    edge_kernel = _make_edge_kernel(n, nch_a, nch_b)

    h, y = _tc_k0(n, x, p["lin_W"], lb, ln_g[0], ln_b[0], cw[0], dis)
    for i in range(6):
        acc = edge_kernel(y, rcw)
        if i < 5:
            h, y = _tc_mid(n, acc, y, h, dis, cb[i], ln_g[i + 1], ln_b[i + 1],
                           cw[i + 1])
        else:
            h = _tc_last(n, acc, y, h, dis, cb[i], ln_g[i + 1], ln_b[i + 1])
    return h


# speedup vs baseline: 1.0918x; 1.0918x over previous
import dataclasses
import functools

import jax
import jax.numpy as jnp
from jax import lax
from jax.experimental import pallas as pl
from jax.experimental.pallas import tpu as pltpu
from jax.experimental.pallas import tpu_sc as plsc

D = 128
NC = 2
NS = 16
NW = NC * NS
LANES = 16
CH = 128
NBUF = 2
NRCW = 4
BR = 1000


def _sc_compiler_params():
    cp = pltpu.CompilerParams()
    if "needs_layout_passes" in pltpu.CompilerParams.__dataclass_fields__:
        cp = dataclasses.replace(cp, needs_layout_passes=False)
    return cp


def _make_deg_kernel(n, nch):
    mesh = plsc.VectorSubcoreMesh(core_axis_name="c", subcore_axis_name="s")

    @functools.partial(
        pl.kernel, mesh=mesh,
        out_type=jax.ShapeDtypeStruct((NW, 1, n), jnp.float32),
        compiler_params=_sc_compiler_params(),
        scratch_types=[
            pltpu.VMEM((nch, CH), jnp.int32),
            pltpu.VMEM((nch, CH), jnp.float32),
            pltpu.VMEM((n,), jnp.float32),
        ],
    )
    def deg_kernel(col_hbm, ew_hbm, out_hbm, colv, ewv, acc):
        wid = lax.axis_index("s") * NC + lax.axis_index("c")
        pltpu.sync_copy(col_hbm.at[wid], colv)
        pltpu.sync_copy(ew_hbm.at[wid], ewv)
        zeros = jnp.zeros((LANES,), jnp.float32)

        @pl.loop(0, n // LANES)
        def _(i):
            acc[pl.ds(i * LANES, LANES)] = zeros

        @pl.loop(0, nch)
        def _(g):
            for k in range(CH // LANES):
                idx = colv[g, pl.ds(k * LANES, LANES)]
                vals = ewv[g, pl.ds(k * LANES, LANES)]
                plsc.addupdate_scatter(acc, [idx], vals)

        pltpu.sync_copy(acc, out_hbm.at[wid, 0])

    return deg_kernel


def _make_edge_kernel(n, nch_a, nch_b):
    rps = (n // (NS * 8)) * 8
    rem = n - NS * rps
    assert rem % 8 == 0 and rem <= CH
    assert min(nch_a, nch_b) >= 4
    full, tail = divmod(rps, CH)
    mesh = plsc.VectorSubcoreMesh(core_axis_name="c", subcore_axis_name="s")

    @functools.partial(
        pl.kernel, mesh=mesh,
        out_type=jax.ShapeDtypeStruct((NC, n, D), jnp.float32),
        compiler_params=_sc_compiler_params(),
        scratch_types=[
            pltpu.VMEM((NRCW * 3, CH), jnp.int32),
            pltpu.VMEM((NBUF, CH, D), jnp.float32),
            pltpu.VMEM_SHARED((n, D), jnp.float32),
            pltpu.SemaphoreType.DMA((NRCW + 2 * NBUF,)),
        ],
    )
    def edge_kernel(y_hbm, rcw_hbm, out_hbm, rcw, rows, acc, sem):
        rsem = sem.at[pl.ds(0, NRCW)]
        gsem = sem.at[pl.ds(NRCW, NBUF)]
        ssem = sem.at[pl.ds(NRCW + NBUF, NBUF)]
        cid = lax.axis_index("c")
        sid = lax.axis_index("s")
        mynch = jnp.where(cid == 0, nch_a, nch_b)
        cbase = jnp.where(cid == 0, sid * nch_a, NS * nch_a + sid * nch_b)

        def r_start(g):
            b = lax.rem(g, NRCW)
            pltpu.async_copy(rcw_hbm.at[cbase + g], rcw.at[pl.ds(b * 3, 3)],
                             rsem.at[b])

        def r_wait(g):
            b = lax.rem(g, NRCW)
            pltpu.make_async_copy(rcw_hbm.at[cbase + g],
                                  rcw.at[pl.ds(b * 3, 3)],
                                  rsem.at[b]).wait()

        def g_start(g):
            b = lax.rem(g, NBUF)
            b5 = lax.rem(g, NRCW)
            pltpu.async_copy(y_hbm.at[rcw.at[b5 * 3]], rows.at[b], gsem.at[b])

        def g_wait(g):
            b = lax.rem(g, NBUF)
            b5 = lax.rem(g, NRCW)
            pltpu.make_async_copy(y_hbm.at[rcw.at[b5 * 3]], rows.at[b],
                                  gsem.at[b]).wait()

        def s_start(g):
            b = lax.rem(g, NBUF)
            b5 = lax.rem(g, NRCW)
            pltpu.async_copy(rows.at[b], acc.at[rcw.at[b5 * 3 + 1]],
                             ssem.at[b], add=True)

        def s_wait(g):
            b = lax.rem(g, NBUF)
            b5 = lax.rem(g, NRCW)
            pltpu.make_async_copy(rows.at[b], acc.at[rcw.at[b5 * 3 + 1]],
                                  ssem.at[b]).wait()

        def scale(g):
            b = lax.rem(g, NBUF)
            ewrow = lax.rem(g, NRCW) * 3 + 2

            @pl.loop(0, CH, unroll=4)
            def _(i):
                ri = jnp.full((LANES,), ewrow, jnp.int32)
                ii = jnp.full((LANES,), i, jnp.int32)
                w = plsc.bitcast(plsc.load_gather(rcw, [ri, ii]),
                                 jnp.float32)
                for j in range(D // LANES):
                    sl = (b, i, pl.ds(j * LANES, LANES))
                    rows[sl] = rows[sl] * w

        zeros = jnp.zeros((LANES,), jnp.float32)

        @pl.loop(0, CH)
        def _(i):
            for j in range(D // LANES):
                rows[0, i, pl.ds(j * LANES, LANES)] = zeros

        base = sid * rps

        @pl.loop(0, full)
        def _(t):
            pltpu.sync_copy(rows.at[0], acc.at[pl.ds(base + t * CH, CH)])

        if tail:
            pltpu.sync_copy(rows.at[0, pl.ds(0, tail)],
                            acc.at[pl.ds(base + full * CH, tail)])
        if rem:
            @pl.when(sid == 0)
            def _():
                pltpu.sync_copy(rows.at[0, pl.ds(0, rem)],
                                acc.at[pl.ds(NS * rps, rem)])

        plsc.subcore_barrier()

        r_start(0)
        r_start(1)

        @pl.loop(0, max(nch_a, nch_b) + 1)
        def _(t0):
            @pl.when(t0 < mynch)
            def _():
                @pl.when(t0 >= NBUF)
                def _():
                    s_wait(t0 - NBUF)

                @pl.when(jnp.logical_and(t0 + 1 >= 2, t0 + 1 < mynch))
                def _():
                    r_start(t0 + 1)

                r_wait(t0)
                g_start(t0)

            @pl.when(jnp.logical_and(t0 >= 1, t0 <= mynch))
            def _():
                t = t0 - 1
                g_wait(t)
                scale(t)
                s_start(t)

        @pl.loop(0, NBUF)
        def _(k):
            s_wait(mynch - NBUF + k)

        plsc.subcore_barrier()

        pltpu.sync_copy(acc.at[pl.ds(base, rps)],
                        out_hbm.at[cid, pl.ds(base, rps)])
        if rem:
            @pl.when(sid == 0)
            def _():
                pltpu.sync_copy(acc.at[pl.ds(NS * rps, rem)],
                                out_hbm.at[cid, pl.ds(NS * rps, rem)])

    return edge_kernel


def _ln_gelu(t, g, b):
    mu = jnp.mean(t, axis=-1, keepdims=True)
    var = jnp.mean((t - mu) ** 2, axis=-1, keepdims=True)
    t = (t - mu) * lax.rsqrt(var + 1e-5) * g + b
    return 0.5 * t * (1.0 + lax.erf(t * 0.7071067811865476))


def _dis_body(degp_ref, dis_ref):
    deg = 1.0 + jnp.sum(degp_ref[...], axis=0, keepdims=True)
    dis_ref[...] = jnp.where(deg > 0, lax.rsqrt(jnp.maximum(deg, 1e-30)), 0.0)


def _k0_body(x_ref, lw_ref, lb_ref, g_ref, b_ref, w1_ref, dis_ref,
             h_ref, y_ref):
    h = jnp.dot(x_ref[...], lw_ref[...],
                preferred_element_type=jnp.float32) + lb_ref[...]
    h = _ln_gelu(h, g_ref[...], b_ref[...])
    h_ref[...] = h
    y_ref[...] = dis_ref[...] * jnp.dot(h, w1_ref[...],
                                        preferred_element_type=jnp.float32)


def _kmid_body(acc_ref, y_ref, hp_ref, dis_ref, cb_ref, g_ref, b_ref, wn_ref,
               h_ref, yo_ref):
    s = acc_ref[0] + acc_ref[1] + y_ref[...]
    conv = dis_ref[...] * s + cb_ref[...]
    h = _ln_gelu(conv, g_ref[...], b_ref[...]) + hp_ref[...]
    h_ref[...] = h
    yo_ref[...] = dis_ref[...] * jnp.dot(h, wn_ref[...],
                                         preferred_element_type=jnp.float32)


def _klast_body(acc_ref, y_ref, hp_ref, dis_ref, cb_ref, g_ref, b_ref, h_ref):
    s = acc_ref[0] + acc_ref[1] + y_ref[...]
    conv = dis_ref[...] * s + cb_ref[...]
    h_ref[...] = _ln_gelu(conv, g_ref[...], b_ref[...]) + hp_ref[...]


def _row_spec(n):
    return pl.BlockSpec((BR, D), lambda g: (g, 0))


_W_SPEC = pl.BlockSpec((D, D), lambda g: (0, 0))
_P_SPEC = pl.BlockSpec((1, D), lambda g: (0, 0))


def _tc_k0(n, x, lw, lb, g0, b0, w1, dis):
    sds = jax.ShapeDtypeStruct((n, D), jnp.float32)
    return pl.pallas_call(
        _k0_body,
        grid=(n // BR,),
        in_specs=[_row_spec(n), _W_SPEC, _P_SPEC, _P_SPEC, _P_SPEC, _W_SPEC,
                  pl.BlockSpec((BR, 1), lambda g: (g, 0))],
        out_specs=[_row_spec(n), _row_spec(n)],
        out_shape=[sds, sds],
    )(x, lw, lb, g0, b0, w1, dis)


def _tc_mid(n, acc, y, hp, dis, cb, g, b, wn):
    sds = jax.ShapeDtypeStruct((n, D), jnp.float32)
    return pl.pallas_call(
        _kmid_body,
        grid=(n // BR,),
        in_specs=[pl.BlockSpec((NC, BR, D), lambda g: (0, g, 0)),
                  _row_spec(n), _row_spec(n),
                  pl.BlockSpec((BR, 1), lambda g: (g, 0)),
                  _P_SPEC, _P_SPEC, _P_SPEC, _W_SPEC],
        out_specs=[_row_spec(n), _row_spec(n)],
        out_shape=[sds, sds],
    )(acc, y, hp, dis, cb, g, b, wn)


def _tc_last(n, acc, y, hp, dis, cb, g, b):
    sds = jax.ShapeDtypeStruct((n, D), jnp.float32)
    return pl.pallas_call(
        _klast_body,
        grid=(n // BR,),
        in_specs=[pl.BlockSpec((NC, BR, D), lambda g: (0, g, 0)),
                  _row_spec(n), _row_spec(n),
                  pl.BlockSpec((BR, 1), lambda g: (g, 0)),
                  _P_SPEC, _P_SPEC, _P_SPEC],
        out_specs=_row_spec(n),
        out_shape=sds,
    )(acc, y, hp, dis, cb, g, b)


def _tc_dis(n, degp):
    return pl.pallas_call(
        _dis_body,
        out_shape=jax.ShapeDtypeStruct((1, n), jnp.float32),
    )(degp)


def kernel(x, edge_index, edge_weight, params):
    n, d = x.shape
    e = edge_weight.shape[0]
    assert d == D and n % LANES == 0 and n % BR == 0

    row = edge_index[0].astype(jnp.int32)
    col = edge_index[1].astype(jnp.int32)
    ew = edge_weight.astype(jnp.float32)

    epw = -(-e // NW)
    epw = -(-epw // CH) * CH
    nchd = epw // CH
    padd = epw * NW - e
    colp = jnp.pad(col, (0, padd)).reshape(NW, nchd, CH)
    ewpd = jnp.pad(ew, (0, padd)).reshape(NW, nchd, CH)
    degp = _make_deg_kernel(n, nchd)(colp, ewpd).reshape(NW, n)
    dis = _tc_dis(n, degp).reshape(n, 1)

    tpp = -(-(-(-e // CH)) // NS)
    nch_a = max(NBUF + 2, round(tpp * 0.28))
    nch_b = tpp - nch_a
    tot = NS * tpp
    pad = tot * CH - e
    rowp = jnp.pad(row, (0, pad)).reshape(tot, 1, CH)
    colp2 = jnp.pad(col, (0, pad)).reshape(tot, 1, CH)
    ewp = jnp.pad(ew, (0, pad)).reshape(tot, 1, CH)
    rcw = jnp.concatenate(
        [rowp, colp2, lax.bitcast_convert_type(ewp, jnp.int32)], axis=1)

    p = params
    lb = p["lin_b"].reshape(1, D)
    ln_g = [p["ln_g"][i].reshape(1, D) for i in range(7)]
    ln_b = [p["ln_b"][i].reshape(1, D) for i in range(7)]
    cb = [p["conv_b"][i].reshape(1, D) for i in range(6)]
    cw = [p["conv_W"][i] for i in range(6)]

    edge_kernel = _make_edge_kernel(n, nch_a, nch_b)

    h, y = _tc_k0(n, x, p["lin_W"], lb, ln_g[0], ln_b[0], cw[0], dis)
    for i in range(6):
        acc = edge_kernel(y, rcw)
        if i < 5:
            h, y = _tc_mid(n, acc, y, h, dis, cb[i], ln_g[i + 1], ln_b[i + 1],
                           cw[i + 1])
        else:
            h = _tc_last(n, acc, y, h, dis, cb[i], ln_g[i + 1], ln_b[i + 1])
    return h


# speedup vs baseline: 1.2367x; 1.1327x over previous
import dataclasses
import functools

import jax
import jax.numpy as jnp
from jax import lax
from jax.experimental import pallas as pl
from jax.experimental.pallas import tpu as pltpu
from jax.experimental.pallas import tpu_sc as plsc

D = 128
NC = 2
NS = 16
NW = NC * NS
LANES = 16
CH = 128
NBUF = 2
NRCW = 4
BR = 1000


def _sc_compiler_params():
    cp = pltpu.CompilerParams()
    if "needs_layout_passes" in pltpu.CompilerParams.__dataclass_fields__:
        cp = dataclasses.replace(cp, needs_layout_passes=False)
    return cp


def _make_deg_kernel(n, nch):
    mesh = plsc.VectorSubcoreMesh(core_axis_name="c", subcore_axis_name="s")

    @functools.partial(
        pl.kernel, mesh=mesh,
        out_type=jax.ShapeDtypeStruct((NW, 1, n), jnp.float32),
        compiler_params=_sc_compiler_params(),
        scratch_types=[
            pltpu.VMEM((nch, CH), jnp.int32),
            pltpu.VMEM((nch, CH), jnp.float32),
            pltpu.VMEM((n,), jnp.float32),
        ],
    )
    def deg_kernel(col_hbm, ew_hbm, out_hbm, colv, ewv, acc):
        wid = lax.axis_index("s") * NC + lax.axis_index("c")
        pltpu.sync_copy(col_hbm.at[wid], colv)
        pltpu.sync_copy(ew_hbm.at[wid], ewv)
        zeros = jnp.zeros((LANES,), jnp.float32)

        @pl.loop(0, n // LANES)
        def _(i):
            acc[pl.ds(i * LANES, LANES)] = zeros

        @pl.loop(0, nch)
        def _(g):
            for k in range(CH // LANES):
                idx = colv[g, pl.ds(k * LANES, LANES)]
                vals = ewv[g, pl.ds(k * LANES, LANES)]
                plsc.addupdate_scatter(acc, [idx], vals)

        pltpu.sync_copy(acc, out_hbm.at[wid, 0])

    return deg_kernel


def _make_edge_kernel(n, nch_a, nch_b):
    rps = (n // (NS * 8)) * 8
    rem = n - NS * rps
    assert rem % 8 == 0 and rem <= CH
    assert min(nch_a, nch_b) >= 4
    full, tail = divmod(rps, CH)
    mesh = plsc.VectorSubcoreMesh(core_axis_name="c", subcore_axis_name="s")

    @functools.partial(
        pl.kernel, mesh=mesh,
        out_type=jax.ShapeDtypeStruct((NC, n, D), jnp.float32),
        compiler_params=_sc_compiler_params(),
        scratch_types=[
            pltpu.VMEM((NRCW * 3, CH), jnp.int32),
            pltpu.VMEM((NBUF, CH, D), jnp.float32),
            pltpu.VMEM_SHARED((n, D), jnp.float32),
            pltpu.SemaphoreType.DMA((NRCW + 2 * NBUF,)),
        ],
    )
    def edge_kernel(y_hbm, rcw_hbm, out_hbm, rcw, rows, acc, sem):
        rsem = sem.at[pl.ds(0, NRCW)]
        gsem = sem.at[pl.ds(NRCW, NBUF)]
        ssem = sem.at[pl.ds(NRCW + NBUF, NBUF)]
        cid = lax.axis_index("c")
        sid = lax.axis_index("s")
        mynch = jnp.where(cid == 0, nch_a, nch_b)
        cbase = jnp.where(cid == 0, sid * nch_a, NS * nch_a + sid * nch_b)

        def r_start(g):
            b = lax.rem(g, NRCW)
            pltpu.async_copy(rcw_hbm.at[cbase + g], rcw.at[pl.ds(b * 3, 3)],
                             rsem.at[b])

        def r_wait(g):
            b = lax.rem(g, NRCW)
            pltpu.make_async_copy(rcw_hbm.at[cbase + g],
                                  rcw.at[pl.ds(b * 3, 3)],
                                  rsem.at[b]).wait()

        def g_start(g):
            b = lax.rem(g, NBUF)
            b5 = lax.rem(g, NRCW)
            pltpu.async_copy(y_hbm.at[rcw.at[b5 * 3]], rows.at[b], gsem.at[b])

        def g_wait(g):
            b = lax.rem(g, NBUF)
            b5 = lax.rem(g, NRCW)
            pltpu.make_async_copy(y_hbm.at[rcw.at[b5 * 3]], rows.at[b],
                                  gsem.at[b]).wait()

        def s_start(g):
            b = lax.rem(g, NBUF)
            b5 = lax.rem(g, NRCW)
            pltpu.async_copy(rows.at[b], acc.at[rcw.at[b5 * 3 + 1]],
                             ssem.at[b], add=True)

        def s_wait(g):
            b = lax.rem(g, NBUF)
            b5 = lax.rem(g, NRCW)
            pltpu.make_async_copy(rows.at[b], acc.at[rcw.at[b5 * 3 + 1]],
                                  ssem.at[b]).wait()

        def scale(g):
            b = lax.rem(g, NBUF)
            ewrow = lax.rem(g, NRCW) * 3 + 2

            @pl.loop(0, CH, unroll=4)
            def _(i):
                ri = jnp.full((LANES,), ewrow, jnp.int32)
                ii = jnp.full((LANES,), i, jnp.int32)
                w = plsc.bitcast(plsc.load_gather(rcw, [ri, ii]),
                                 jnp.float32)
                for j in range(D // LANES):
                    sl = (b, i, pl.ds(j * LANES, LANES))
                    rows[sl] = rows[sl] * w

        zeros = jnp.zeros((LANES,), jnp.float32)

        @pl.loop(0, CH)
        def _(i):
            for j in range(D // LANES):
                rows[0, i, pl.ds(j * LANES, LANES)] = zeros

        base = sid * rps

        @pl.loop(0, full)
        def _(t):
            pltpu.sync_copy(rows.at[0], acc.at[pl.ds(base + t * CH, CH)])

        if tail:
            pltpu.sync_copy(rows.at[0, pl.ds(0, tail)],
                            acc.at[pl.ds(base + full * CH, tail)])
        if rem:
            @pl.when(sid == 0)
            def _():
                pltpu.sync_copy(rows.at[0, pl.ds(0, rem)],
                                acc.at[pl.ds(NS * rps, rem)])

        plsc.subcore_barrier()

        r_start(0)
        r_start(1)

        @pl.loop(0, max(nch_a, nch_b) + 1)
        def _(t0):
            @pl.when(t0 < mynch)
            def _():
                @pl.when(t0 >= NBUF)
                def _():
                    s_wait(t0 - NBUF)

                @pl.when(jnp.logical_and(t0 + 1 >= 2, t0 + 1 < mynch))
                def _():
                    r_start(t0 + 1)

                r_wait(t0)
                g_start(t0)

            @pl.when(jnp.logical_and(t0 >= 1, t0 <= mynch))
            def _():
                t = t0 - 1
                g_wait(t)
                scale(t)
                s_start(t)

        @pl.loop(0, NBUF)
        def _(k):
            s_wait(mynch - NBUF + k)

        plsc.subcore_barrier()

        pltpu.sync_copy(acc.at[pl.ds(base, rps)],
                        out_hbm.at[cid, pl.ds(base, rps)])
        if rem:
            @pl.when(sid == 0)
            def _():
                pltpu.sync_copy(acc.at[pl.ds(NS * rps, rem)],
                                out_hbm.at[cid, pl.ds(NS * rps, rem)])

    return edge_kernel


def _ln_gelu(t, g, b):
    mu = jnp.mean(t, axis=-1, keepdims=True)
    var = jnp.mean((t - mu) ** 2, axis=-1, keepdims=True)
    t = (t - mu) * lax.rsqrt(var + 1e-5) * g + b
    return 0.5 * t * (1.0 + lax.erf(t * 0.7071067811865476))


def _dis_body(degp_ref, dis_ref):
    deg = 1.0 + jnp.sum(degp_ref[...], axis=0, keepdims=True)
    dis_ref[...] = jnp.where(deg > 0, lax.rsqrt(jnp.maximum(deg, 1e-30)), 0.0)


def _k0_body(x_ref, lw_ref, lb_ref, g_ref, b_ref, w1_ref, dis_ref,
             h_ref, y_ref):
    h = jnp.dot(x_ref[...], lw_ref[...],
                preferred_element_type=jnp.float32) + lb_ref[...]
    h = _ln_gelu(h, g_ref[...], b_ref[...])
    h_ref[...] = h
    y_ref[...] = dis_ref[...] * jnp.dot(h, w1_ref[...],
                                        preferred_element_type=jnp.float32)


def _kmid_body(acc_ref, y_ref, hp_ref, dis_ref, cb_ref, g_ref, b_ref, wn_ref,
               h_ref, yo_ref):
    s = acc_ref[0] + acc_ref[1] + y_ref[...]
    conv = dis_ref[...] * s + cb_ref[...]
    h = _ln_gelu(conv, g_ref[...], b_ref[...]) + hp_ref[...]
    h_ref[...] = h
    yo_ref[...] = dis_ref[...] * jnp.dot(h, wn_ref[...],
                                         preferred_element_type=jnp.float32)


def _klast_body(acc_ref, y_ref, hp_ref, dis_ref, cb_ref, g_ref, b_ref, h_ref):
    s = acc_ref[0] + acc_ref[1] + y_ref[...]
    conv = dis_ref[...] * s + cb_ref[...]
    h_ref[...] = _ln_gelu(conv, g_ref[...], b_ref[...]) + hp_ref[...]


def _row_spec(n):
    return pl.BlockSpec((BR, D), lambda g: (g, 0))


_W_SPEC = pl.BlockSpec((D, D), lambda g: (0, 0))
_P_SPEC = pl.BlockSpec((1, D), lambda g: (0, 0))


def _tc_k0(n, x, lw, lb, g0, b0, w1, dis):
    sds = jax.ShapeDtypeStruct((n, D), jnp.float32)
    return pl.pallas_call(
        _k0_body,
        grid=(n // BR,),
        in_specs=[_row_spec(n), _W_SPEC, _P_SPEC, _P_SPEC, _P_SPEC, _W_SPEC,
                  pl.BlockSpec((BR, 1), lambda g: (g, 0))],
        out_specs=[_row_spec(n), _row_spec(n)],
        out_shape=[sds, sds],
    )(x, lw, lb, g0, b0, w1, dis)


def _tc_mid(n, acc, y, hp, dis, cb, g, b, wn):
    sds = jax.ShapeDtypeStruct((n, D), jnp.float32)
    return pl.pallas_call(
        _kmid_body,
        grid=(n // BR,),
        in_specs=[pl.BlockSpec((NC, BR, D), lambda g: (0, g, 0)),
                  _row_spec(n), _row_spec(n),
                  pl.BlockSpec((BR, 1), lambda g: (g, 0)),
                  _P_SPEC, _P_SPEC, _P_SPEC, _W_SPEC],
        out_specs=[_row_spec(n), _row_spec(n)],
        out_shape=[sds, sds],
    )(acc, y, hp, dis, cb, g, b, wn)


def _tc_last(n, acc, y, hp, dis, cb, g, b):
    sds = jax.ShapeDtypeStruct((n, D), jnp.float32)
    return pl.pallas_call(
        _klast_body,
        grid=(n // BR,),
        in_specs=[pl.BlockSpec((NC, BR, D), lambda g: (0, g, 0)),
                  _row_spec(n), _row_spec(n),
                  pl.BlockSpec((BR, 1), lambda g: (g, 0)),
                  _P_SPEC, _P_SPEC, _P_SPEC],
        out_specs=_row_spec(n),
        out_shape=sds,
    )(acc, y, hp, dis, cb, g, b)


def _tc_dis(n, degp):
    return pl.pallas_call(
        _dis_body,
        out_shape=jax.ShapeDtypeStruct((1, n), jnp.float32),
    )(degp)


def kernel(x, edge_index, edge_weight, params):
    n, d = x.shape
    e = edge_weight.shape[0]
    assert d == D and n % LANES == 0 and n % BR == 0

    row = edge_index[0].astype(jnp.int32)
    col = edge_index[1].astype(jnp.int32)
    ew = edge_weight.astype(jnp.float32)

    epw = -(-e // NW)
    epw = -(-epw // CH) * CH
    nchd = epw // CH
    padd = epw * NW - e
    colp = jnp.pad(col, (0, padd)).reshape(NW, nchd, CH)
    ewpd = jnp.pad(ew, (0, padd)).reshape(NW, nchd, CH)
    degp = _make_deg_kernel(n, nchd)(colp, ewpd).reshape(NW, n)
    dis = _tc_dis(n, degp).reshape(n, 1)

    tpp = -(-(-(-e // CH)) // NS)
    nch_a = max(NBUF + 2, round(tpp * 0.40))
    nch_b = tpp - nch_a
    tot = NS * tpp
    pad = tot * CH - e
    rowp = jnp.pad(row, (0, pad)).reshape(tot, 1, CH)
    colp2 = jnp.pad(col, (0, pad)).reshape(tot, 1, CH)
    ewp = jnp.pad(ew, (0, pad)).reshape(tot, 1, CH)
    rcw = jnp.concatenate(
        [rowp, colp2, lax.bitcast_convert_type(ewp, jnp.int32)], axis=1)

    p = params
    lb = p["lin_b"].reshape(1, D)
    ln_g = [p["ln_g"][i].reshape(1, D) for i in range(7)]
    ln_b = [p["ln_b"][i].reshape(1, D) for i in range(7)]
    cb = [p["conv_b"][i].reshape(1, D) for i in range(6)]
    cw = [p["conv_W"][i] for i in range(6)]

    edge_kernel = _make_edge_kernel(n, nch_a, nch_b)

    h, y = _tc_k0(n, x, p["lin_W"], lb, ln_g[0], ln_b[0], cw[0], dis)
    for i in range(6):
        acc = edge_kernel(y, rcw)
        if i < 5:
            h, y = _tc_mid(n, acc, y, h, dis, cb[i], ln_g[i + 1], ln_b[i + 1],
                           cw[i + 1])
        else:
            h = _tc_last(n, acc, y, h, dis, cb[i], ln_g[i + 1], ln_b[i + 1])
    return h


# speedup vs baseline: 1.2906x; 1.0436x over previous
import dataclasses
import functools

import jax
import jax.numpy as jnp
from jax import lax
from jax.experimental import pallas as pl
from jax.experimental.pallas import tpu as pltpu
from jax.experimental.pallas import tpu_sc as plsc

D = 128
NC = 2
NS = 16
NW = NC * NS
LANES = 16
CH = 128
NBUF = 2
NRCW = 4
BR = 1000


def _sc_compiler_params():
    cp = pltpu.CompilerParams()
    if "needs_layout_passes" in pltpu.CompilerParams.__dataclass_fields__:
        cp = dataclasses.replace(cp, needs_layout_passes=False)
    return cp


def _make_deg_kernel(n, nch):
    mesh = plsc.VectorSubcoreMesh(core_axis_name="c", subcore_axis_name="s")

    @functools.partial(
        pl.kernel, mesh=mesh,
        out_type=jax.ShapeDtypeStruct((NW, 1, n), jnp.float32),
        compiler_params=_sc_compiler_params(),
        scratch_types=[
            pltpu.VMEM((nch, CH), jnp.int32),
            pltpu.VMEM((nch, CH), jnp.float32),
            pltpu.VMEM((n,), jnp.float32),
        ],
    )
    def deg_kernel(col_hbm, ew_hbm, out_hbm, colv, ewv, acc):
        wid = lax.axis_index("s") * NC + lax.axis_index("c")
        pltpu.sync_copy(col_hbm.at[wid], colv)
        pltpu.sync_copy(ew_hbm.at[wid], ewv)
        zeros = jnp.zeros((LANES,), jnp.float32)

        @pl.loop(0, n // LANES)
        def _(i):
            acc[pl.ds(i * LANES, LANES)] = zeros

        @pl.loop(0, nch)
        def _(g):
            for k in range(CH // LANES):
                idx = colv[g, pl.ds(k * LANES, LANES)]
                vals = ewv[g, pl.ds(k * LANES, LANES)]
                plsc.addupdate_scatter(acc, [idx], vals)

        pltpu.sync_copy(acc, out_hbm.at[wid, 0])

    return deg_kernel


def _make_edge_kernel(n, nch_a, nch_b):
    rps = (n // (NS * 8)) * 8
    rem = n - NS * rps
    assert rem % 8 == 0 and rem <= CH
    assert min(nch_a, nch_b) >= 4
    full, tail = divmod(rps, CH)
    mesh = plsc.VectorSubcoreMesh(core_axis_name="c", subcore_axis_name="s")

    @functools.partial(
        pl.kernel, mesh=mesh,
        out_type=jax.ShapeDtypeStruct((NC, n, D), jnp.float32),
        compiler_params=_sc_compiler_params(),
        scratch_types=[
            pltpu.VMEM((NRCW * 3, CH), jnp.int32),
            pltpu.VMEM((NBUF, CH, D), jnp.float32),
            pltpu.VMEM_SHARED((n, D), jnp.float32),
            pltpu.SemaphoreType.DMA((NRCW + 2 * NBUF,)),
        ],
    )
    def edge_kernel(y_hbm, rcw_hbm, out_hbm, rcw, rows, acc, sem):
        rsem = sem.at[pl.ds(0, NRCW)]
        gsem = sem.at[pl.ds(NRCW, NBUF)]
        ssem = sem.at[pl.ds(NRCW + NBUF, NBUF)]
        cid = lax.axis_index("c")
        sid = lax.axis_index("s")
        mynch = jnp.where(cid == 0, nch_a, nch_b)
        cbase = jnp.where(cid == 0, sid * nch_a, NS * nch_a + sid * nch_b)

        def r_start(g):
            b = lax.rem(g, NRCW)
            pltpu.async_copy(rcw_hbm.at[cbase + g], rcw.at[pl.ds(b * 3, 3)],
                             rsem.at[b])

        def r_wait(g):
            b = lax.rem(g, NRCW)
            pltpu.make_async_copy(rcw_hbm.at[cbase + g],
                                  rcw.at[pl.ds(b * 3, 3)],
                                  rsem.at[b]).wait()

        def g_start(g):
            b = lax.rem(g, NBUF)
            b5 = lax.rem(g, NRCW)
            pltpu.async_copy(y_hbm.at[rcw.at[b5 * 3]], rows.at[b], gsem.at[b])

        def g_wait(g):
            b = lax.rem(g, NBUF)
            b5 = lax.rem(g, NRCW)
            pltpu.make_async_copy(y_hbm.at[rcw.at[b5 * 3]], rows.at[b],
                                  gsem.at[b]).wait()

        def s_start(g):
            b = lax.rem(g, NBUF)
            b5 = lax.rem(g, NRCW)
            pltpu.async_copy(rows.at[b], acc.at[rcw.at[b5 * 3 + 1]],
                             ssem.at[b], add=True)

        def s_wait(g):
            b = lax.rem(g, NBUF)
            b5 = lax.rem(g, NRCW)
            pltpu.make_async_copy(rows.at[b], acc.at[rcw.at[b5 * 3 + 1]],
                                  ssem.at[b]).wait()

        def scale(g):
            b = lax.rem(g, NBUF)
            ewrow = lax.rem(g, NRCW) * 3 + 2

            @pl.loop(0, CH, unroll=4)
            def _(i):
                ri = jnp.full((LANES,), ewrow, jnp.int32)
                ii = jnp.full((LANES,), i, jnp.int32)
                w = plsc.bitcast(plsc.load_gather(rcw, [ri, ii]),
                                 jnp.float32)
                for j in range(D // LANES):
                    sl = (b, i, pl.ds(j * LANES, LANES))
                    rows[sl] = rows[sl] * w

        zeros = jnp.zeros((LANES,), jnp.float32)

        @pl.loop(0, CH)
        def _(i):
            for j in range(D // LANES):
                rows[0, i, pl.ds(j * LANES, LANES)] = zeros

        base = sid * rps

        @pl.loop(0, full)
        def _(t):
            pltpu.sync_copy(rows.at[0], acc.at[pl.ds(base + t * CH, CH)])

        if tail:
            pltpu.sync_copy(rows.at[0, pl.ds(0, tail)],
                            acc.at[pl.ds(base + full * CH, tail)])
        if rem:
            @pl.when(sid == 0)
            def _():
                pltpu.sync_copy(rows.at[0, pl.ds(0, rem)],
                                acc.at[pl.ds(NS * rps, rem)])

        plsc.subcore_barrier()

        r_start(0)
        r_start(1)

        @pl.loop(0, max(nch_a, nch_b) + 1)
        def _(t0):
            @pl.when(t0 < mynch)
            def _():
                @pl.when(t0 >= NBUF)
                def _():
                    s_wait(t0 - NBUF)

                @pl.when(jnp.logical_and(t0 + 1 >= 2, t0 + 1 < mynch))
                def _():
                    r_start(t0 + 1)

                r_wait(t0)
                g_start(t0)

            @pl.when(jnp.logical_and(t0 >= 1, t0 <= mynch))
            def _():
                t = t0 - 1
                g_wait(t)
                scale(t)
                s_start(t)

        @pl.loop(0, NBUF)
        def _(k):
            s_wait(mynch - NBUF + k)

        plsc.subcore_barrier()

        pltpu.sync_copy(acc.at[pl.ds(base, rps)],
                        out_hbm.at[cid, pl.ds(base, rps)])
        if rem:
            @pl.when(sid == 0)
            def _():
                pltpu.sync_copy(acc.at[pl.ds(NS * rps, rem)],
                                out_hbm.at[cid, pl.ds(NS * rps, rem)])

    return edge_kernel


def _ln_gelu(t, g, b):
    mu = jnp.mean(t, axis=-1, keepdims=True)
    var = jnp.mean((t - mu) ** 2, axis=-1, keepdims=True)
    t = (t - mu) * lax.rsqrt(var + 1e-5) * g + b
    return 0.5 * t * (1.0 + lax.erf(t * 0.7071067811865476))


def _dis_body(degp_ref, dis_ref):
    deg = 1.0 + jnp.sum(degp_ref[...], axis=0, keepdims=True)
    dis_ref[...] = jnp.where(deg > 0, lax.rsqrt(jnp.maximum(deg, 1e-30)), 0.0)


def _k0_body(x_ref, lw_ref, lb_ref, g_ref, b_ref, w1_ref, dis_ref,
             h_ref, y_ref):
    h = jnp.dot(x_ref[...], lw_ref[...],
                preferred_element_type=jnp.float32) + lb_ref[...]
    h = _ln_gelu(h, g_ref[...], b_ref[...])
    h_ref[...] = h
    y_ref[...] = dis_ref[...] * jnp.dot(h, w1_ref[...],
                                        preferred_element_type=jnp.float32)


def _kmid_body(acc_ref, y_ref, hp_ref, dis_ref, cb_ref, g_ref, b_ref, wn_ref,
               h_ref, yo_ref):
    s = acc_ref[0] + acc_ref[1] + y_ref[...]
    conv = dis_ref[...] * s + cb_ref[...]
    h = _ln_gelu(conv, g_ref[...], b_ref[...]) + hp_ref[...]
    h_ref[...] = h
    yo_ref[...] = dis_ref[...] * jnp.dot(h, wn_ref[...],
                                         preferred_element_type=jnp.float32)


def _klast_body(acc_ref, y_ref, hp_ref, dis_ref, cb_ref, g_ref, b_ref, h_ref):
    s = acc_ref[0] + acc_ref[1] + y_ref[...]
    conv = dis_ref[...] * s + cb_ref[...]
    h_ref[...] = _ln_gelu(conv, g_ref[...], b_ref[...]) + hp_ref[...]


def _row_spec(n):
    return pl.BlockSpec((BR, D), lambda g: (g, 0))


_W_SPEC = pl.BlockSpec((D, D), lambda g: (0, 0))
_P_SPEC = pl.BlockSpec((1, D), lambda g: (0, 0))


def _tc_k0(n, x, lw, lb, g0, b0, w1, dis):
    sds = jax.ShapeDtypeStruct((n, D), jnp.float32)
    return pl.pallas_call(
        _k0_body,
        grid=(n // BR,),
        in_specs=[_row_spec(n), _W_SPEC, _P_SPEC, _P_SPEC, _P_SPEC, _W_SPEC,
                  pl.BlockSpec((BR, 1), lambda g: (g, 0))],
        out_specs=[_row_spec(n), _row_spec(n)],
        out_shape=[sds, sds],
    )(x, lw, lb, g0, b0, w1, dis)


def _tc_mid(n, acc, y, hp, dis, cb, g, b, wn):
    sds = jax.ShapeDtypeStruct((n, D), jnp.float32)
    return pl.pallas_call(
        _kmid_body,
        grid=(n // BR,),
        in_specs=[pl.BlockSpec((NC, BR, D), lambda g: (0, g, 0)),
                  _row_spec(n), _row_spec(n),
                  pl.BlockSpec((BR, 1), lambda g: (g, 0)),
                  _P_SPEC, _P_SPEC, _P_SPEC, _W_SPEC],
        out_specs=[_row_spec(n), _row_spec(n)],
        out_shape=[sds, sds],
    )(acc, y, hp, dis, cb, g, b, wn)


def _tc_last(n, acc, y, hp, dis, cb, g, b):
    sds = jax.ShapeDtypeStruct((n, D), jnp.float32)
    return pl.pallas_call(
        _klast_body,
        grid=(n // BR,),
        in_specs=[pl.BlockSpec((NC, BR, D), lambda g: (0, g, 0)),
                  _row_spec(n), _row_spec(n),
                  pl.BlockSpec((BR, 1), lambda g: (g, 0)),
                  _P_SPEC, _P_SPEC, _P_SPEC],
        out_specs=_row_spec(n),
        out_shape=sds,
    )(acc, y, hp, dis, cb, g, b)


def _tc_dis(n, degp):
    return pl.pallas_call(
        _dis_body,
        out_shape=jax.ShapeDtypeStruct((1, n), jnp.float32),
    )(degp)


def kernel(x, edge_index, edge_weight, params):
    n, d = x.shape
    e = edge_weight.shape[0]
    assert d == D and n % LANES == 0 and n % BR == 0

    row = edge_index[0].astype(jnp.int32)
    col = edge_index[1].astype(jnp.int32)
    ew = edge_weight.astype(jnp.float32)

    epw = -(-e // NW)
    epw = -(-epw // CH) * CH
    nchd = epw // CH
    padd = epw * NW - e
    colp = jnp.pad(col, (0, padd)).reshape(NW, nchd, CH)
    ewpd = jnp.pad(ew, (0, padd)).reshape(NW, nchd, CH)
    degp = _make_deg_kernel(n, nchd)(colp, ewpd).reshape(NW, n)
    dis = _tc_dis(n, degp).reshape(n, 1)

    tpp = -(-(-(-e // CH)) // NS)
    nch_a = max(NBUF + 2, round(tpp * 0.44))
    nch_b = tpp - nch_a
    tot = NS * tpp
    pad = tot * CH - e
    rowp = jnp.pad(row, (0, pad)).reshape(tot, 1, CH)
    colp2 = jnp.pad(col, (0, pad)).reshape(tot, 1, CH)
    ewp = jnp.pad(ew, (0, pad)).reshape(tot, 1, CH)
    rcw = jnp.concatenate(
        [rowp, colp2, lax.bitcast_convert_type(ewp, jnp.int32)], axis=1)

    p = params
    lb = p["lin_b"].reshape(1, D)
    ln_g = [p["ln_g"][i].reshape(1, D) for i in range(7)]
    ln_b = [p["ln_b"][i].reshape(1, D) for i in range(7)]
    cb = [p["conv_b"][i].reshape(1, D) for i in range(6)]
    cw = [p["conv_W"][i] for i in range(6)]

    edge_kernel = _make_edge_kernel(n, nch_a, nch_b)

    h, y = _tc_k0(n, x, p["lin_W"], lb, ln_g[0], ln_b[0], cw[0], dis)
    for i in range(6):
        acc = edge_kernel(y, rcw)
        if i < 5:
            h, y = _tc_mid(n, acc, y, h, dis, cb[i], ln_g[i + 1], ln_b[i + 1],
                           cw[i + 1])
        else:
            h = _tc_last(n, acc, y, h, dis, cb[i], ln_g[i + 1], ln_b[i + 1])
    return h


# speedup vs baseline: 1.3231x; 1.0252x over previous
import dataclasses
import functools

import jax
import jax.numpy as jnp
from jax import lax
from jax.experimental import pallas as pl
from jax.experimental.pallas import tpu as pltpu
from jax.experimental.pallas import tpu_sc as plsc

D = 128
NC = 2
NS = 16
NW = NC * NS
LANES = 16
CH = 128
NBUF = 2
NRCW = 4
BR = 1000


def _sc_compiler_params():
    cp = pltpu.CompilerParams()
    if "needs_layout_passes" in pltpu.CompilerParams.__dataclass_fields__:
        cp = dataclasses.replace(cp, needs_layout_passes=False)
    return cp


def _make_deg_kernel(n, nch):
    mesh = plsc.VectorSubcoreMesh(core_axis_name="c", subcore_axis_name="s")

    @functools.partial(
        pl.kernel, mesh=mesh,
        out_type=jax.ShapeDtypeStruct((NW, 1, n), jnp.float32),
        compiler_params=_sc_compiler_params(),
        scratch_types=[
            pltpu.VMEM((nch, CH), jnp.int32),
            pltpu.VMEM((nch, CH), jnp.float32),
            pltpu.VMEM((n,), jnp.float32),
        ],
    )
    def deg_kernel(col_hbm, ew_hbm, out_hbm, colv, ewv, acc):
        wid = lax.axis_index("s") * NC + lax.axis_index("c")
        pltpu.sync_copy(col_hbm.at[wid], colv)
        pltpu.sync_copy(ew_hbm.at[wid], ewv)
        zeros = jnp.zeros((LANES,), jnp.float32)

        @pl.loop(0, n // LANES)
        def _(i):
            acc[pl.ds(i * LANES, LANES)] = zeros

        @pl.loop(0, nch)
        def _(g):
            for k in range(CH // LANES):
                idx = colv[g, pl.ds(k * LANES, LANES)]
                vals = ewv[g, pl.ds(k * LANES, LANES)]
                plsc.addupdate_scatter(acc, [idx], vals)

        pltpu.sync_copy(acc, out_hbm.at[wid, 0])

    return deg_kernel


def _make_edge_kernel(n, nch_a, nch_b):
    rps = (n // (NS * 8)) * 8
    rem = n - NS * rps
    assert rem % 8 == 0 and rem <= CH
    assert min(nch_a, nch_b) >= 4
    full, tail = divmod(rps, CH)
    mesh = plsc.VectorSubcoreMesh(core_axis_name="c", subcore_axis_name="s")

    @functools.partial(
        pl.kernel, mesh=mesh,
        out_type=jax.ShapeDtypeStruct((NC, n, D), jnp.float32),
        compiler_params=_sc_compiler_params(),
        scratch_types=[
            pltpu.VMEM((NRCW * 3, CH), jnp.int32),
            pltpu.VMEM((NBUF, CH, D), jnp.float32),
            pltpu.VMEM_SHARED((n, D), jnp.float32),
            pltpu.SemaphoreType.DMA((NRCW + 2 * NBUF,)),
        ],
    )
    def edge_kernel(y_hbm, rcw_hbm, out_hbm, rcw, rows, acc, sem):
        rsem = sem.at[pl.ds(0, NRCW)]
        gsem = sem.at[pl.ds(NRCW, NBUF)]
        ssem = sem.at[pl.ds(NRCW + NBUF, NBUF)]
        cid = lax.axis_index("c")
        sid = lax.axis_index("s")
        mynch = jnp.where(cid == 0, nch_a, nch_b)
        cbase = jnp.where(cid == 0, sid * nch_a, NS * nch_a + sid * nch_b)

        def r_start(g):
            b = lax.rem(g, NRCW)
            pltpu.async_copy(rcw_hbm.at[cbase + g], rcw.at[pl.ds(b * 3, 3)],
                             rsem.at[b])

        def r_wait(g):
            b = lax.rem(g, NRCW)
            pltpu.make_async_copy(rcw_hbm.at[cbase + g],
                                  rcw.at[pl.ds(b * 3, 3)],
                                  rsem.at[b]).wait()

        def g_start(g):
            b = lax.rem(g, NBUF)
            b5 = lax.rem(g, NRCW)
            pltpu.async_copy(y_hbm.at[rcw.at[b5 * 3]], rows.at[b], gsem.at[b])

        def g_wait(g):
            b = lax.rem(g, NBUF)
            b5 = lax.rem(g, NRCW)
            pltpu.make_async_copy(y_hbm.at[rcw.at[b5 * 3]], rows.at[b],
                                  gsem.at[b]).wait()

        def s_start(g):
            b = lax.rem(g, NBUF)
            b5 = lax.rem(g, NRCW)
            pltpu.async_copy(rows.at[b], acc.at[rcw.at[b5 * 3 + 1]],
                             ssem.at[b], add=True)

        def s_wait(g):
            b = lax.rem(g, NBUF)
            b5 = lax.rem(g, NRCW)
            pltpu.make_async_copy(rows.at[b], acc.at[rcw.at[b5 * 3 + 1]],
                                  ssem.at[b]).wait()

        def scale(g):
            b = lax.rem(g, NBUF)
            ewrow = lax.rem(g, NRCW) * 3 + 2

            @pl.loop(0, CH, unroll=4)
            def _(i):
                ri = jnp.full((LANES,), ewrow, jnp.int32)
                ii = jnp.full((LANES,), i, jnp.int32)
                w = plsc.bitcast(plsc.load_gather(rcw, [ri, ii]),
                                 jnp.float32)
                for j in range(D // LANES):
                    sl = (b, i, pl.ds(j * LANES, LANES))
                    rows[sl] = rows[sl] * w

        zeros = jnp.zeros((LANES,), jnp.float32)

        @pl.loop(0, CH)
        def _(i):
            for j in range(D // LANES):
                rows[0, i, pl.ds(j * LANES, LANES)] = zeros

        base = sid * rps

        @pl.loop(0, full)
        def _(t):
            pltpu.sync_copy(rows.at[0], acc.at[pl.ds(base + t * CH, CH)])

        if tail:
            pltpu.sync_copy(rows.at[0, pl.ds(0, tail)],
                            acc.at[pl.ds(base + full * CH, tail)])
        if rem:
            @pl.when(sid == 0)
            def _():
                pltpu.sync_copy(rows.at[0, pl.ds(0, rem)],
                                acc.at[pl.ds(NS * rps, rem)])

        plsc.subcore_barrier()

        r_start(0)
        r_start(1)

        @pl.loop(0, max(nch_a, nch_b) + 1)
        def _(t0):
            @pl.when(t0 < mynch)
            def _():
                @pl.when(t0 >= NBUF)
                def _():
                    s_wait(t0 - NBUF)

                @pl.when(jnp.logical_and(t0 + 1 >= 2, t0 + 1 < mynch))
                def _():
                    r_start(t0 + 1)

                r_wait(t0)
                g_start(t0)

            @pl.when(jnp.logical_and(t0 >= 1, t0 <= mynch))
            def _():
                t = t0 - 1
                g_wait(t)
                scale(t)
                s_start(t)

        @pl.loop(0, NBUF)
        def _(k):
            s_wait(mynch - NBUF + k)

        plsc.subcore_barrier()

        pltpu.sync_copy(acc.at[pl.ds(base, rps)],
                        out_hbm.at[cid, pl.ds(base, rps)])
        if rem:
            @pl.when(sid == 0)
            def _():
                pltpu.sync_copy(acc.at[pl.ds(NS * rps, rem)],
                                out_hbm.at[cid, pl.ds(NS * rps, rem)])

    return edge_kernel


def _ln_gelu(t, g, b):
    mu = jnp.mean(t, axis=-1, keepdims=True)
    var = jnp.mean((t - mu) ** 2, axis=-1, keepdims=True)
    t = (t - mu) * lax.rsqrt(var + 1e-5) * g + b
    return 0.5 * t * (1.0 + lax.erf(t * 0.7071067811865476))


def _dis_body(degp_ref, dis_ref):
    deg = 1.0 + jnp.sum(degp_ref[...], axis=0, keepdims=True)
    dis_ref[...] = jnp.where(deg > 0, lax.rsqrt(jnp.maximum(deg, 1e-30)), 0.0)


def _k0_body(x_ref, lw_ref, lb_ref, g_ref, b_ref, w1_ref, dis_ref,
             h_ref, y_ref):
    h = jnp.dot(x_ref[...], lw_ref[...],
                preferred_element_type=jnp.float32) + lb_ref[...]
    h = _ln_gelu(h, g_ref[...], b_ref[...])
    h_ref[...] = h
    y_ref[...] = dis_ref[...] * jnp.dot(h, w1_ref[...],
                                        preferred_element_type=jnp.float32)


def _kmid_body(acc_ref, y_ref, hp_ref, dis_ref, cb_ref, g_ref, b_ref, wn_ref,
               h_ref, yo_ref):
    s = acc_ref[0] + acc_ref[1] + y_ref[...]
    conv = dis_ref[...] * s + cb_ref[...]
    h = _ln_gelu(conv, g_ref[...], b_ref[...]) + hp_ref[...]
    h_ref[...] = h
    yo_ref[...] = dis_ref[...] * jnp.dot(h, wn_ref[...],
                                         preferred_element_type=jnp.float32)


def _klast_body(acc_ref, y_ref, hp_ref, dis_ref, cb_ref, g_ref, b_ref, h_ref):
    s = acc_ref[0] + acc_ref[1] + y_ref[...]
    conv = dis_ref[...] * s + cb_ref[...]
    h_ref[...] = _ln_gelu(conv, g_ref[...], b_ref[...]) + hp_ref[...]


def _row_spec(n):
    return pl.BlockSpec((BR, D), lambda g: (g, 0))


_W_SPEC = pl.BlockSpec((D, D), lambda g: (0, 0))
_P_SPEC = pl.BlockSpec((1, D), lambda g: (0, 0))


def _tc_k0(n, x, lw, lb, g0, b0, w1, dis):
    sds = jax.ShapeDtypeStruct((n, D), jnp.float32)
    return pl.pallas_call(
        _k0_body,
        grid=(n // BR,),
        in_specs=[_row_spec(n), _W_SPEC, _P_SPEC, _P_SPEC, _P_SPEC, _W_SPEC,
                  pl.BlockSpec((BR, 1), lambda g: (g, 0))],
        out_specs=[_row_spec(n), _row_spec(n)],
        out_shape=[sds, sds],
    )(x, lw, lb, g0, b0, w1, dis)


def _tc_mid(n, acc, y, hp, dis, cb, g, b, wn):
    sds = jax.ShapeDtypeStruct((n, D), jnp.float32)
    return pl.pallas_call(
        _kmid_body,
        grid=(n // BR,),
        in_specs=[pl.BlockSpec((NC, BR, D), lambda g: (0, g, 0)),
                  _row_spec(n), _row_spec(n),
                  pl.BlockSpec((BR, 1), lambda g: (g, 0)),
                  _P_SPEC, _P_SPEC, _P_SPEC, _W_SPEC],
        out_specs=[_row_spec(n), _row_spec(n)],
        out_shape=[sds, sds],
    )(acc, y, hp, dis, cb, g, b, wn)


def _tc_last(n, acc, y, hp, dis, cb, g, b):
    sds = jax.ShapeDtypeStruct((n, D), jnp.float32)
    return pl.pallas_call(
        _klast_body,
        grid=(n // BR,),
        in_specs=[pl.BlockSpec((NC, BR, D), lambda g: (0, g, 0)),
                  _row_spec(n), _row_spec(n),
                  pl.BlockSpec((BR, 1), lambda g: (g, 0)),
                  _P_SPEC, _P_SPEC, _P_SPEC],
        out_specs=_row_spec(n),
        out_shape=sds,
    )(acc, y, hp, dis, cb, g, b)


def _tc_dis(n, degp):
    return pl.pallas_call(
        _dis_body,
        out_shape=jax.ShapeDtypeStruct((1, n), jnp.float32),
    )(degp)


def kernel(x, edge_index, edge_weight, params):
    n, d = x.shape
    e = edge_weight.shape[0]
    assert d == D and n % LANES == 0 and n % BR == 0

    row = edge_index[0].astype(jnp.int32)
    col = edge_index[1].astype(jnp.int32)
    ew = edge_weight.astype(jnp.float32)

    epw = -(-e // NW)
    epw = -(-epw // CH) * CH
    nchd = epw // CH
    padd = epw * NW - e
    colp = jnp.pad(col, (0, padd)).reshape(NW, nchd, CH)
    ewpd = jnp.pad(ew, (0, padd)).reshape(NW, nchd, CH)
    degp = _make_deg_kernel(n, nchd)(colp, ewpd).reshape(NW, n)
    dis = _tc_dis(n, degp).reshape(n, 1)

    tpp = -(-(-(-e // CH)) // NS)
    nch_a = max(NBUF + 2, round(tpp * 0.48))
    nch_b = tpp - nch_a
    tot = NS * tpp
    pad = tot * CH - e
    rowp = jnp.pad(row, (0, pad)).reshape(tot, 1, CH)
    colp2 = jnp.pad(col, (0, pad)).reshape(tot, 1, CH)
    ewp = jnp.pad(ew, (0, pad)).reshape(tot, 1, CH)
    rcw = jnp.concatenate(
        [rowp, colp2, lax.bitcast_convert_type(ewp, jnp.int32)], axis=1)

    p = params
    lb = p["lin_b"].reshape(1, D)
    ln_g = [p["ln_g"][i].reshape(1, D) for i in range(7)]
    ln_b = [p["ln_b"][i].reshape(1, D) for i in range(7)]
    cb = [p["conv_b"][i].reshape(1, D) for i in range(6)]
    cw = [p["conv_W"][i] for i in range(6)]

    edge_kernel = _make_edge_kernel(n, nch_a, nch_b)

    h, y = _tc_k0(n, x, p["lin_W"], lb, ln_g[0], ln_b[0], cw[0], dis)
    for i in range(6):
        acc = edge_kernel(y, rcw)
        if i < 5:
            h, y = _tc_mid(n, acc, y, h, dis, cb[i], ln_g[i + 1], ln_b[i + 1],
                           cw[i + 1])
        else:
            h = _tc_last(n, acc, y, h, dis, cb[i], ln_g[i + 1], ln_b[i + 1])
    return h


# speedup vs baseline: 1.3382x; 1.0114x over previous
import dataclasses
import functools

import jax
import jax.numpy as jnp
from jax import lax
from jax.experimental import pallas as pl
from jax.experimental.pallas import tpu as pltpu
from jax.experimental.pallas import tpu_sc as plsc

D = 128
NC = 2
NS = 16
NW = NC * NS
LANES = 16
CH = 128
NBUF = 2
NRCW = 4
BR = 1000


def _sc_compiler_params():
    cp = pltpu.CompilerParams()
    if "needs_layout_passes" in pltpu.CompilerParams.__dataclass_fields__:
        cp = dataclasses.replace(cp, needs_layout_passes=False)
    return cp


def _make_deg_kernel(n, nch):
    mesh = plsc.VectorSubcoreMesh(core_axis_name="c", subcore_axis_name="s")

    @functools.partial(
        pl.kernel, mesh=mesh,
        out_type=jax.ShapeDtypeStruct((NW, 1, n), jnp.float32),
        compiler_params=_sc_compiler_params(),
        scratch_types=[
            pltpu.VMEM((nch, CH), jnp.int32),
            pltpu.VMEM((nch, CH), jnp.float32),
            pltpu.VMEM((n,), jnp.float32),
        ],
    )
    def deg_kernel(col_hbm, ew_hbm, out_hbm, colv, ewv, acc):
        wid = lax.axis_index("s") * NC + lax.axis_index("c")
        pltpu.sync_copy(col_hbm.at[wid], colv)
        pltpu.sync_copy(ew_hbm.at[wid], ewv)
        zeros = jnp.zeros((LANES,), jnp.float32)

        @pl.loop(0, n // LANES)
        def _(i):
            acc[pl.ds(i * LANES, LANES)] = zeros

        @pl.loop(0, nch)
        def _(g):
            for k in range(CH // LANES):
                idx = colv[g, pl.ds(k * LANES, LANES)]
                vals = ewv[g, pl.ds(k * LANES, LANES)]
                plsc.addupdate_scatter(acc, [idx], vals)

        pltpu.sync_copy(acc, out_hbm.at[wid, 0])

    return deg_kernel


def _make_edge_kernel(n, nch_a, nch_b):
    rps = (n // (NS * 8)) * 8
    rem = n - NS * rps
    assert rem % 8 == 0 and rem <= CH
    assert min(nch_a, nch_b) >= 4
    full, tail = divmod(rps, CH)
    mesh = plsc.VectorSubcoreMesh(core_axis_name="c", subcore_axis_name="s")

    @functools.partial(
        pl.kernel, mesh=mesh,
        out_type=jax.ShapeDtypeStruct((NC, n, D), jnp.float32),
        compiler_params=_sc_compiler_params(),
        scratch_types=[
            pltpu.VMEM((NRCW * 3, CH), jnp.int32),
            pltpu.VMEM((NBUF, CH, D), jnp.float32),
            pltpu.VMEM_SHARED((n, D), jnp.float32),
            pltpu.SemaphoreType.DMA((NRCW + 2 * NBUF,)),
        ],
    )
    def edge_kernel(y_hbm, rcw_hbm, out_hbm, rcw, rows, acc, sem):
        rsem = sem.at[pl.ds(0, NRCW)]
        gsem = sem.at[pl.ds(NRCW, NBUF)]
        ssem = sem.at[pl.ds(NRCW + NBUF, NBUF)]
        cid = lax.axis_index("c")
        sid = lax.axis_index("s")
        mynch = jnp.where(cid == 0, nch_a, nch_b)
        cbase = jnp.where(cid == 0, sid * nch_a, NS * nch_a + sid * nch_b)

        def r_start(g):
            b = lax.rem(g, NRCW)
            pltpu.async_copy(rcw_hbm.at[cbase + g], rcw.at[pl.ds(b * 3, 3)],
                             rsem.at[b])

        def r_wait(g):
            b = lax.rem(g, NRCW)
            pltpu.make_async_copy(rcw_hbm.at[cbase + g],
                                  rcw.at[pl.ds(b * 3, 3)],
                                  rsem.at[b]).wait()

        def g_start(g):
            b = lax.rem(g, NBUF)
            b5 = lax.rem(g, NRCW)
            pltpu.async_copy(y_hbm.at[rcw.at[b5 * 3]], rows.at[b], gsem.at[b])

        def g_wait(g):
            b = lax.rem(g, NBUF)
            b5 = lax.rem(g, NRCW)
            pltpu.make_async_copy(y_hbm.at[rcw.at[b5 * 3]], rows.at[b],
                                  gsem.at[b]).wait()

        def s_start(g):
            b = lax.rem(g, NBUF)
            b5 = lax.rem(g, NRCW)
            pltpu.async_copy(rows.at[b], acc.at[rcw.at[b5 * 3 + 1]],
                             ssem.at[b], add=True)

        def s_wait(g):
            b = lax.rem(g, NBUF)
            b5 = lax.rem(g, NRCW)
            pltpu.make_async_copy(rows.at[b], acc.at[rcw.at[b5 * 3 + 1]],
                                  ssem.at[b]).wait()

        def scale(g):
            b = lax.rem(g, NBUF)
            ewrow = lax.rem(g, NRCW) * 3 + 2

            @pl.loop(0, CH, unroll=4)
            def _(i):
                ri = jnp.full((LANES,), ewrow, jnp.int32)
                ii = jnp.full((LANES,), i, jnp.int32)
                w = plsc.bitcast(plsc.load_gather(rcw, [ri, ii]),
                                 jnp.float32)
                for j in range(D // LANES):
                    sl = (b, i, pl.ds(j * LANES, LANES))
                    rows[sl] = rows[sl] * w

        zeros = jnp.zeros((LANES,), jnp.float32)

        @pl.loop(0, CH)
        def _(i):
            for j in range(D // LANES):
                rows[0, i, pl.ds(j * LANES, LANES)] = zeros

        base = sid * rps

        @pl.loop(0, full)
        def _(t):
            pltpu.sync_copy(rows.at[0], acc.at[pl.ds(base + t * CH, CH)])

        if tail:
            pltpu.sync_copy(rows.at[0, pl.ds(0, tail)],
                            acc.at[pl.ds(base + full * CH, tail)])
        if rem:
            @pl.when(sid == 0)
            def _():
                pltpu.sync_copy(rows.at[0, pl.ds(0, rem)],
                                acc.at[pl.ds(NS * rps, rem)])

        plsc.subcore_barrier()

        r_start(0)
        r_start(1)

        @pl.loop(0, max(nch_a, nch_b) + 1)
        def _(t0):
            @pl.when(t0 < mynch)
            def _():
                @pl.when(t0 >= NBUF)
                def _():
                    s_wait(t0 - NBUF)

                @pl.when(jnp.logical_and(t0 + 1 >= 2, t0 + 1 < mynch))
                def _():
                    r_start(t0 + 1)

                r_wait(t0)
                g_start(t0)

            @pl.when(jnp.logical_and(t0 >= 1, t0 <= mynch))
            def _():
                t = t0 - 1
                g_wait(t)
                scale(t)
                s_start(t)

        @pl.loop(0, NBUF)
        def _(k):
            s_wait(mynch - NBUF + k)

        plsc.subcore_barrier()

        pltpu.sync_copy(acc.at[pl.ds(base, rps)],
                        out_hbm.at[cid, pl.ds(base, rps)])
        if rem:
            @pl.when(sid == 0)
            def _():
                pltpu.sync_copy(acc.at[pl.ds(NS * rps, rem)],
                                out_hbm.at[cid, pl.ds(NS * rps, rem)])

    return edge_kernel


def _ln_gelu(t, g, b):
    mu = jnp.mean(t, axis=-1, keepdims=True)
    var = jnp.mean((t - mu) ** 2, axis=-1, keepdims=True)
    t = (t - mu) * lax.rsqrt(var + 1e-5) * g + b
    return 0.5 * t * (1.0 + lax.erf(t * 0.7071067811865476))


def _dis_body(degp_ref, dis_ref):
    deg = 1.0 + jnp.sum(degp_ref[...], axis=0, keepdims=True)
    dis_ref[...] = jnp.where(deg > 0, lax.rsqrt(jnp.maximum(deg, 1e-30)), 0.0)


def _k0_body(x_ref, lw_ref, lb_ref, g_ref, b_ref, w1_ref, dis_ref,
             h_ref, y_ref):
    h = jnp.dot(x_ref[...], lw_ref[...],
                preferred_element_type=jnp.float32) + lb_ref[...]
    h = _ln_gelu(h, g_ref[...], b_ref[...])
    h_ref[...] = h
    y_ref[...] = dis_ref[...] * jnp.dot(h, w1_ref[...],
                                        preferred_element_type=jnp.float32)


def _kmid_body(acc_ref, y_ref, hp_ref, dis_ref, cb_ref, g_ref, b_ref, wn_ref,
               h_ref, yo_ref):
    s = acc_ref[0] + acc_ref[1] + y_ref[...]
    conv = dis_ref[...] * s + cb_ref[...]
    h = _ln_gelu(conv, g_ref[...], b_ref[...]) + hp_ref[...]
    h_ref[...] = h
    yo_ref[...] = dis_ref[...] * jnp.dot(h, wn_ref[...],
                                         preferred_element_type=jnp.float32)


def _klast_body(acc_ref, y_ref, hp_ref, dis_ref, cb_ref, g_ref, b_ref, h_ref):
    s = acc_ref[0] + acc_ref[1] + y_ref[...]
    conv = dis_ref[...] * s + cb_ref[...]
    h_ref[...] = _ln_gelu(conv, g_ref[...], b_ref[...]) + hp_ref[...]


def _row_spec(n):
    return pl.BlockSpec((BR, D), lambda g: (g, 0))


_W_SPEC = pl.BlockSpec((D, D), lambda g: (0, 0))
_P_SPEC = pl.BlockSpec((1, D), lambda g: (0, 0))


def _tc_k0(n, x, lw, lb, g0, b0, w1, dis):
    sds = jax.ShapeDtypeStruct((n, D), jnp.float32)
    return pl.pallas_call(
        _k0_body,
        grid=(n // BR,),
        in_specs=[_row_spec(n), _W_SPEC, _P_SPEC, _P_SPEC, _P_SPEC, _W_SPEC,
                  pl.BlockSpec((BR, 1), lambda g: (g, 0))],
        out_specs=[_row_spec(n), _row_spec(n)],
        out_shape=[sds, sds],
    )(x, lw, lb, g0, b0, w1, dis)


def _tc_mid(n, acc, y, hp, dis, cb, g, b, wn):
    sds = jax.ShapeDtypeStruct((n, D), jnp.float32)
    return pl.pallas_call(
        _kmid_body,
        grid=(n // BR,),
        in_specs=[pl.BlockSpec((NC, BR, D), lambda g: (0, g, 0)),
                  _row_spec(n), _row_spec(n),
                  pl.BlockSpec((BR, 1), lambda g: (g, 0)),
                  _P_SPEC, _P_SPEC, _P_SPEC, _W_SPEC],
        out_specs=[_row_spec(n), _row_spec(n)],
        out_shape=[sds, sds],
    )(acc, y, hp, dis, cb, g, b, wn)


def _tc_last(n, acc, y, hp, dis, cb, g, b):
    sds = jax.ShapeDtypeStruct((n, D), jnp.float32)
    return pl.pallas_call(
        _klast_body,
        grid=(n // BR,),
        in_specs=[pl.BlockSpec((NC, BR, D), lambda g: (0, g, 0)),
                  _row_spec(n), _row_spec(n),
                  pl.BlockSpec((BR, 1), lambda g: (g, 0)),
                  _P_SPEC, _P_SPEC, _P_SPEC],
        out_specs=_row_spec(n),
        out_shape=sds,
    )(acc, y, hp, dis, cb, g, b)


def _tc_dis(n, degp):
    return pl.pallas_call(
        _dis_body,
        out_shape=jax.ShapeDtypeStruct((1, n), jnp.float32),
    )(degp)


def kernel(x, edge_index, edge_weight, params):
    n, d = x.shape
    e = edge_weight.shape[0]
    assert d == D and n % LANES == 0 and n % BR == 0

    row = edge_index[0].astype(jnp.int32)
    col = edge_index[1].astype(jnp.int32)
    ew = edge_weight.astype(jnp.float32)

    epw = -(-e // NW)
    epw = -(-epw // CH) * CH
    nchd = epw // CH
    padd = epw * NW - e
    colp = jnp.pad(col, (0, padd)).reshape(NW, nchd, CH)
    ewpd = jnp.pad(ew, (0, padd)).reshape(NW, nchd, CH)
    degp = _make_deg_kernel(n, nchd)(colp, ewpd).reshape(NW, n)
    dis = _tc_dis(n, degp).reshape(n, 1)

    tpp = -(-(-(-e // CH)) // NS)
    nch_a = max(NBUF + 2, round(tpp * 0.50))
    nch_b = tpp - nch_a
    tot = NS * tpp
    pad = tot * CH - e
    rowp = jnp.pad(row, (0, pad)).reshape(tot, 1, CH)
    colp2 = jnp.pad(col, (0, pad)).reshape(tot, 1, CH)
    ewp = jnp.pad(ew, (0, pad)).reshape(tot, 1, CH)
    rcw = jnp.concatenate(
        [rowp, colp2, lax.bitcast_convert_type(ewp, jnp.int32)], axis=1)

    p = params
    lb = p["lin_b"].reshape(1, D)
    ln_g = [p["ln_g"][i].reshape(1, D) for i in range(7)]
    ln_b = [p["ln_b"][i].reshape(1, D) for i in range(7)]
    cb = [p["conv_b"][i].reshape(1, D) for i in range(6)]
    cw = [p["conv_W"][i] for i in range(6)]

    edge_kernel = _make_edge_kernel(n, nch_a, nch_b)

    h, y = _tc_k0(n, x, p["lin_W"], lb, ln_g[0], ln_b[0], cw[0], dis)
    for i in range(6):
        acc = edge_kernel(y, rcw)
        if i < 5:
            h, y = _tc_mid(n, acc, y, h, dis, cb[i], ln_g[i + 1], ln_b[i + 1],
                           cw[i + 1])
        else:
            h = _tc_last(n, acc, y, h, dis, cb[i], ln_g[i + 1], ln_b[i + 1])
    return h
